# Initial kernel scaffold; baseline (speedup 1.0000x reference)
import jax
import jax.numpy as jnp
from jax.experimental import pallas as pl


def kernel(x, edge_index, W_l1, b_l1, W_r1, W_l2, b_l2, W_r2):
    raise NotImplementedError("write your pallas kernel here")


# speedup vs baseline: 6.0914x; 6.0914x over previous
import functools

import jax
import jax.numpy as jnp
from jax import lax
from jax.experimental import pallas as pl
from jax.experimental.pallas import tpu as pltpu
from jax.experimental.pallas import tpu_sc as plsc

NC = 2
NS = 16
NW = NC * NS
K = 80


@functools.lru_cache(maxsize=None)
def _make_sc_scatter(n, e, d):
    assert e % (NW * K) == 0 and n % K == 0 and d % 16 == 0
    epw = e // NW
    nch = epw // K
    nrc = n // K

    mesh = plsc.VectorSubcoreMesh(core_axis_name="c", subcore_axis_name="s")

    @functools.partial(
        pl.kernel,
        out_type=jax.ShapeDtypeStruct((NC, n, d), jnp.float32),
        mesh=mesh,
        scratch_types=[
            pltpu.VMEM((K,), jnp.int32),
            pltpu.VMEM((K,), jnp.int32),
            pltpu.VMEM((K, d), jnp.float32),
            pltpu.VMEM_SHARED((n, d), jnp.float32),
            pltpu.SemaphoreType.DMA,
        ],
        compiler_params=pltpu.CompilerParams(use_tc_tiling_on_sc=False),
    )
    def sc_kernel(rows_hbm, src_hbm, dst_hbm, part_hbm,
                  src_v, dst_v, rows_v, acc, sem):
        c = lax.axis_index("c")
        s = lax.axis_index("s")
        wid = s * NC + c

        def zrow(i, carry):
            for j in range(d // 16):
                rows_v[i, pl.ds(j * 16, 16)] = jnp.zeros((16,), jnp.float32)
            return carry
        lax.fori_loop(0, K, zrow, 0)

        def zacc(i, carry):
            t = s + i * NS
            @pl.when(t < nrc)
            def _():
                pltpu.sync_copy(rows_v, acc.at[pl.ds(t * K, K)])
            return carry
        lax.fori_loop(0, (nrc + NS - 1) // NS, zacc, 0)
        plsc.subcore_barrier()

        def chunk(i, carry):
            eb = wid * epw + i * K
            pltpu.sync_copy(src_hbm.at[pl.ds(eb, K)], src_v)
            pltpu.async_copy(rows_hbm.at[src_v], rows_v, sem).wait()
            pltpu.sync_copy(dst_hbm.at[pl.ds(eb, K)], dst_v)
            pltpu.sync_copy(rows_v, acc.at[dst_v], add=True)
            return carry
        lax.fori_loop(0, nch, chunk, 0)
        plsc.subcore_barrier()

        def cout(i, carry):
            t = s + i * NS
            @pl.when(t < nrc)
            def _():
                pltpu.sync_copy(acc.at[pl.ds(t * K, K)],
                                part_hbm.at[c, pl.ds(t * K, K)])
            return carry
        lax.fori_loop(0, (nrc + NS - 1) // NS, cout, 0)

    return sc_kernel


@functools.lru_cache(maxsize=None)
def _make_tc1(n, f_in, hid, f_out, dpad, r):
    grid = n // r

    def body(part, x, wl1, bl1, wr1, wl2, wr2, bl2, p, r2, inv):
        a = part[0] + part[1]
        cnt = a[:, f_in:f_in + 1]
        iv = 1.0 / jnp.maximum(cnt, 1.0)
        mean = a[:, :f_in] * iv
        h = jnp.maximum(
            jnp.dot(mean, wl1[...], preferred_element_type=jnp.float32)
            + bl1[...]
            + jnp.dot(x[...], wr1[...], preferred_element_type=jnp.float32),
            0.0)
        p[...] = jnp.dot(h, wl2[...], preferred_element_type=jnp.float32)
        r2[...] = (jnp.dot(h, wr2[...], preferred_element_type=jnp.float32)
                   + bl2[...])
        inv[...] = iv

    return pl.pallas_call(
        body,
        grid=(grid,),
        in_specs=[
            pl.BlockSpec((NC, r, dpad), lambda i: (0, i, 0)),
            pl.BlockSpec((r, f_in), lambda i: (i, 0)),
            pl.BlockSpec((f_in, hid), lambda i: (0, 0)),
            pl.BlockSpec((1, hid), lambda i: (0, 0)),
            pl.BlockSpec((f_in, hid), lambda i: (0, 0)),
            pl.BlockSpec((hid, f_out), lambda i: (0, 0)),
            pl.BlockSpec((hid, f_out), lambda i: (0, 0)),
            pl.BlockSpec((1, f_out), lambda i: (0, 0)),
        ],
        out_specs=[
            pl.BlockSpec((r, f_out), lambda i: (i, 0)),
            pl.BlockSpec((r, f_out), lambda i: (i, 0)),
            pl.BlockSpec((r, 1), lambda i: (i, 0)),
        ],
        out_shape=[
            jax.ShapeDtypeStruct((n, f_out), jnp.float32),
            jax.ShapeDtypeStruct((n, f_out), jnp.float32),
            jax.ShapeDtypeStruct((n, 1), jnp.float32),
        ],
    )


@functools.lru_cache(maxsize=None)
def _make_tc2(n, f_out, r):
    grid = n // r

    def body(part, inv, r2, out):
        out[...] = (part[0] + part[1]) * inv[...] + r2[...]

    return pl.pallas_call(
        body,
        grid=(grid,),
        in_specs=[
            pl.BlockSpec((NC, r, f_out), lambda i: (0, i, 0)),
            pl.BlockSpec((r, 1), lambda i: (i, 0)),
            pl.BlockSpec((r, f_out), lambda i: (i, 0)),
        ],
        out_specs=pl.BlockSpec((r, f_out), lambda i: (i, 0)),
        out_shape=jax.ShapeDtypeStruct((n, f_out), jnp.float32),
    )


def kernel(x, edge_index, W_l1, b_l1, W_r1, W_l2, b_l2, W_r2):
    n, f_in = x.shape
    e = edge_index.shape[1]
    hid = W_l1.shape[1]
    f_out = W_l2.shape[1]
    dpad = ((f_in + 1 + 15) // 16) * 16

    src = edge_index[0]
    dst = edge_index[1]
    xpad = jnp.concatenate(
        [x, jnp.ones((n, 1), jnp.float32),
         jnp.zeros((n, dpad - f_in - 1), jnp.float32)], axis=1)

    part1 = _make_sc_scatter(n, e, dpad)(xpad, src, dst)
    p, r2, inv = _make_tc1(n, f_in, hid, f_out, dpad, 400)(
        part1, x, W_l1, b_l1.reshape(1, hid), W_r1, W_l2, W_r2,
        b_l2.reshape(1, f_out))
    part2 = _make_sc_scatter(n, e, f_out)(p, src, dst)
    out = _make_tc2(n, f_out, 400)(part2, inv, r2)
    return out


# speedup vs baseline: 13.0446x; 2.1415x over previous
import functools

import jax
import jax.numpy as jnp
from jax import lax
from jax.experimental import pallas as pl
from jax.experimental.pallas import tpu as pltpu
from jax.experimental.pallas import tpu_sc as plsc

NC = 2
NS = 16
NW = NC * NS
K = 80


SPMEM_BUDGET = 2097151 * 4


@functools.lru_cache(maxsize=None)
def _make_sc_scatter(n, e, d):
    assert e % (NW * K) == 0 and n % K == 0 and d % 16 == 0
    epw = e // NW
    nch = epw // K
    nrc = n // K

    nbuf = 2
    for cand in (4,):
        if n * d * 4 + NS * (cand * K * d * 4 + 2 * cand * 2 * K * 4) \
                < SPMEM_BUDGET:
            nbuf = cand
    ir = 2 * nbuf
    assert nch > ir

    mesh = plsc.VectorSubcoreMesh(core_axis_name="c", subcore_axis_name="s")

    @functools.partial(
        pl.kernel,
        out_type=jax.ShapeDtypeStruct((NC, n, d), jnp.float32),
        mesh=mesh,
        scratch_types=[
            *[pltpu.VMEM((2, K), jnp.int32) for _ in range(ir)],
            *[pltpu.VMEM((K, d), jnp.float32) for _ in range(nbuf)],
            pltpu.VMEM_SHARED((n, d), jnp.float32),
            *[pltpu.SemaphoreType.DMA for _ in range(ir + nbuf)],
        ],
        compiler_params=pltpu.CompilerParams(use_tc_tiling_on_sc=False),
    )
    def sc_kernel(rows_hbm, edges_hbm, part_hbm, *scratch):
        ix = scratch[:ir]
        rows_v = scratch[ir:ir + nbuf]
        acc = scratch[ir + nbuf]
        isem = scratch[ir + nbuf + 1:ir + nbuf + 1 + ir]
        gsem = scratch[ir + nbuf + 1 + ir:]
        c = lax.axis_index("c")
        s = lax.axis_index("s")
        wid = s * NC + c
        ch0 = wid * nch

        def ixload(i, a):
            return pltpu.make_async_copy(edges_hbm.at[ch0 + i], ix[a], isem[a])

        def gather(b, a):
            return pltpu.make_async_copy(
                rows_hbm.at[ix[a].at[0]], rows_v[b], gsem[b])

        def zrow(i, carry):
            for j in range(d // 16):
                rows_v[0][i, pl.ds(j * 16, 16)] = jnp.zeros((16,), jnp.float32)
            return carry
        lax.fori_loop(0, K, zrow, 0)

        for a in range(ir):
            ixload(a, a).start()

        def zacc(i, carry):
            t = s + i * NS
            @pl.when(t < nrc)
            def _():
                pltpu.sync_copy(rows_v[0], acc.at[pl.ds(t * K, K)])
            return carry
        lax.fori_loop(0, (nrc + NS - 1) // NS, zacc, 0)
        plsc.subcore_barrier()

        for q in range(nbuf):
            ixload(q, q).wait()
            gather(q, q).start()

        def step(i, q, tail):
            b, a = q % nbuf, q % ir
            gather(b, a).wait()
            pltpu.sync_copy(rows_v[b], acc.at[ix[a].at[1]], add=True)

            def refill():
                ixload(i + ir, a).start()
            def advance():
                a2 = (q + nbuf) % ir
                ixload(i + nbuf, a2).wait()
                gather(b, a2).start()
            if tail:
                if i + ir < nch:
                    refill()
                if i + nbuf < nch:
                    advance()
            else:
                pl.when(i + ir < nch)(refill)
                pl.when(i + nbuf < nch)(advance)

        def outer(j, carry):
            for q in range(ir):
                step(j * ir + q, q, False)
            return carry
        lax.fori_loop(0, nch // ir, outer, 0)
        for i in range((nch // ir) * ir, nch):
            step(i, i % ir, True)
        plsc.subcore_barrier()

        def cout(i, carry):
            t = s + i * NS
            @pl.when(t < nrc)
            def _():
                pltpu.sync_copy(acc.at[pl.ds(t * K, K)],
                                part_hbm.at[c, pl.ds(t * K, K)])
            return carry
        lax.fori_loop(0, (nrc + NS - 1) // NS, cout, 0)

    return sc_kernel


@functools.lru_cache(maxsize=None)
def _make_tc1(n, f_in, hid, f_out, dpad, r):
    grid = n // r

    def body(part, x, wl1, bl1, wr1, wl2, wr2, bl2, p, r2, inv):
        a = part[0] + part[1]
        cnt = a[:, f_in:f_in + 1]
        iv = 1.0 / jnp.maximum(cnt, 1.0)
        mean = a[:, :f_in] * iv
        h = jnp.maximum(
            jnp.dot(mean, wl1[...], preferred_element_type=jnp.float32)
            + bl1[...]
            + jnp.dot(x[...], wr1[...], preferred_element_type=jnp.float32),
            0.0)
        p[...] = jnp.dot(h, wl2[...], preferred_element_type=jnp.float32)
        r2[...] = (jnp.dot(h, wr2[...], preferred_element_type=jnp.float32)
                   + bl2[...])
        inv[...] = iv

    return pl.pallas_call(
        body,
        grid=(grid,),
        in_specs=[
            pl.BlockSpec((NC, r, dpad), lambda i: (0, i, 0)),
            pl.BlockSpec((r, f_in), lambda i: (i, 0)),
            pl.BlockSpec((f_in, hid), lambda i: (0, 0)),
            pl.BlockSpec((1, hid), lambda i: (0, 0)),
            pl.BlockSpec((f_in, hid), lambda i: (0, 0)),
            pl.BlockSpec((hid, f_out), lambda i: (0, 0)),
            pl.BlockSpec((hid, f_out), lambda i: (0, 0)),
            pl.BlockSpec((1, f_out), lambda i: (0, 0)),
        ],
        out_specs=[
            pl.BlockSpec((r, f_out), lambda i: (i, 0)),
            pl.BlockSpec((r, f_out), lambda i: (i, 0)),
            pl.BlockSpec((r, 1), lambda i: (i, 0)),
        ],
        out_shape=[
            jax.ShapeDtypeStruct((n, f_out), jnp.float32),
            jax.ShapeDtypeStruct((n, f_out), jnp.float32),
            jax.ShapeDtypeStruct((n, 1), jnp.float32),
        ],
    )


@functools.lru_cache(maxsize=None)
def _make_tc2(n, f_out, r):
    grid = n // r

    def body(part, inv, r2, out):
        out[...] = (part[0] + part[1]) * inv[...] + r2[...]

    return pl.pallas_call(
        body,
        grid=(grid,),
        in_specs=[
            pl.BlockSpec((NC, r, f_out), lambda i: (0, i, 0)),
            pl.BlockSpec((r, 1), lambda i: (i, 0)),
            pl.BlockSpec((r, f_out), lambda i: (i, 0)),
        ],
        out_specs=pl.BlockSpec((r, f_out), lambda i: (i, 0)),
        out_shape=jax.ShapeDtypeStruct((n, f_out), jnp.float32),
    )


def kernel(x, edge_index, W_l1, b_l1, W_r1, W_l2, b_l2, W_r2):
    n, f_in = x.shape
    e = edge_index.shape[1]
    hid = W_l1.shape[1]
    f_out = W_l2.shape[1]
    dpad = ((f_in + 1 + 15) // 16) * 16

    edges = edge_index.reshape(2, e // K, K).transpose(1, 0, 2)
    xpad = jnp.concatenate(
        [x, jnp.ones((n, 1), jnp.float32),
         jnp.zeros((n, dpad - f_in - 1), jnp.float32)], axis=1)

    part1 = _make_sc_scatter(n, e, dpad)(xpad, edges)
    p, r2, inv = _make_tc1(n, f_in, hid, f_out, dpad, 400)(
        part1, x, W_l1, b_l1.reshape(1, hid), W_r1, W_l2, W_r2,
        b_l2.reshape(1, f_out))
    part2 = _make_sc_scatter(n, e, f_out)(p, edges)
    out = _make_tc2(n, f_out, 400)(part2, inv, r2)
    return out


# speedup vs baseline: 15.6069x; 1.1964x over previous
import functools

import jax
import jax.numpy as jnp
from jax import lax
from jax.experimental import pallas as pl
from jax.experimental.pallas import tpu as pltpu
from jax.experimental.pallas import tpu_sc as plsc

NC = 2
NS = 16
NW = NC * NS
K = 80


SPMEM_BUDGET = 2097151 * 4


@functools.lru_cache(maxsize=None)
def _make_sc_scatter(n, e, d, with_cnt):
    assert e % (NW * K) == 0 and n % K == 0 and d % 16 == 0
    epw = e // NW
    nch = epw // K
    nrc = n // K

    nbuf = 2
    for cand in (3, 4):
        if n * d * 4 + NS * (cand * K * d * 4 + 2 * cand * 2 * K * 4
                             + 8 * K) + n * 8 < SPMEM_BUDGET:
            nbuf = cand
    ir = 2 * nbuf
    assert nch > ir

    mesh = plsc.VectorSubcoreMesh(core_axis_name="c", subcore_axis_name="s")

    out_type = [jax.ShapeDtypeStruct((NC, n, d), jnp.float32)]
    cnt_scratch = []
    if with_cnt:
        out_type.append(jax.ShapeDtypeStruct((NC, n, 1), jnp.float32))
        cnt_scratch = [
            pltpu.VMEM((K, 1), jnp.float32),
            pltpu.VMEM((K, 1), jnp.float32),
            pltpu.VMEM_SHARED((n, 1), jnp.float32),
        ]

    @functools.partial(
        pl.kernel,
        out_type=tuple(out_type) if with_cnt else out_type[0],
        mesh=mesh,
        scratch_types=[
            *[pltpu.VMEM((2, K), jnp.int32) for _ in range(ir)],
            *[pltpu.VMEM((K, d), jnp.float32) for _ in range(nbuf)],
            pltpu.VMEM_SHARED((n, d), jnp.float32),
            *cnt_scratch,
            *[pltpu.SemaphoreType.DMA for _ in range(ir + nbuf)],
        ],
        compiler_params=pltpu.CompilerParams(use_tc_tiling_on_sc=False),
    )
    def sc_kernel(*args):
        it = iter(args)
        rows_hbm = next(it)
        edges_hbm = next(it)
        cinit_hbm = next(it) if with_cnt else None
        part_hbm = next(it)
        pcnt_hbm = next(it) if with_cnt else None
        ix = [next(it) for _ in range(ir)]
        rows_v = [next(it) for _ in range(nbuf)]
        acc = next(it)
        if with_cnt:
            ones_v, zcol_v, acc1 = next(it), next(it), next(it)
        isem = [next(it) for _ in range(ir)]
        gsem = [next(it) for _ in range(nbuf)]

        c = lax.axis_index("c")
        s = lax.axis_index("s")
        wid = s * NC + c
        ch0 = wid * nch

        def ixload(i, a):
            return pltpu.make_async_copy(edges_hbm.at[ch0 + i], ix[a], isem[a])

        def gather(b, a):
            return pltpu.make_async_copy(
                rows_hbm.at[ix[a].at[0]], rows_v[b], gsem[b])

        def zrow(i, carry):
            for j in range(d // 16):
                rows_v[0][i, pl.ds(j * 16, 16)] = jnp.zeros((16,), jnp.float32)
            return carry
        lax.fori_loop(0, K, zrow, 0)

        for a in range(ir):
            ixload(a, a).start()
        if with_cnt:
            pltpu.sync_copy(cinit_hbm.at[pl.ds(0, K)], ones_v)
            pltpu.sync_copy(cinit_hbm.at[pl.ds(K, K)], zcol_v)

        def zacc(i, carry):
            t = s + i * NS
            @pl.when(t < nrc)
            def _():
                pltpu.sync_copy(rows_v[0], acc.at[pl.ds(t * K, K)])
                if with_cnt:
                    pltpu.sync_copy(zcol_v, acc1.at[pl.ds(t * K, K)])
            return carry
        lax.fori_loop(0, (nrc + NS - 1) // NS, zacc, 0)
        plsc.subcore_barrier()

        for q in range(nbuf):
            ixload(q, q).wait()
            gather(q, q).start()

        def step(i, q, tail):
            b, a = q % nbuf, q % ir
            gather(b, a).wait()
            pltpu.sync_copy(rows_v[b], acc.at[ix[a].at[1]], add=True)
            if with_cnt:
                pltpu.sync_copy(ones_v, acc1.at[ix[a].at[1]], add=True)

            def refill():
                ixload(i + ir, a).start()
            def advance():
                a2 = (q + nbuf) % ir
                ixload(i + nbuf, a2).wait()
                gather(b, a2).start()
            if tail:
                if i + ir < nch:
                    refill()
                if i + nbuf < nch:
                    advance()
            else:
                pl.when(i + ir < nch)(refill)
                pl.when(i + nbuf < nch)(advance)

        def outer(j, carry):
            for q in range(ir):
                step(j * ir + q, q, False)
            return carry
        lax.fori_loop(0, nch // ir, outer, 0)
        for i in range((nch // ir) * ir, nch):
            step(i, i % ir, True)
        plsc.subcore_barrier()

        def cout(i, carry):
            t = s + i * NS
            @pl.when(t < nrc)
            def _():
                pltpu.sync_copy(acc.at[pl.ds(t * K, K)],
                                part_hbm.at[c, pl.ds(t * K, K)])
                if with_cnt:
                    pltpu.sync_copy(acc1.at[pl.ds(t * K, K)],
                                    pcnt_hbm.at[c, pl.ds(t * K, K)])
            return carry
        lax.fori_loop(0, (nrc + NS - 1) // NS, cout, 0)

    return sc_kernel


@functools.lru_cache(maxsize=None)
def _make_tc1(n, f_in, hid, f_out, r):
    grid = n // r

    def body(part, pcnt, x, wl1, bl1, wr1, wl2, wr2, bl2, p, r2, inv):
        a = part[0] + part[1]
        cnt = pcnt[0] + pcnt[1]
        iv = 1.0 / jnp.maximum(cnt, 1.0)
        mean = a * iv
        h = jnp.maximum(
            jnp.dot(mean, wl1[...], preferred_element_type=jnp.float32)
            + bl1[...]
            + jnp.dot(x[...], wr1[...], preferred_element_type=jnp.float32),
            0.0)
        p[...] = jnp.dot(h, wl2[...], preferred_element_type=jnp.float32)
        r2[...] = (jnp.dot(h, wr2[...], preferred_element_type=jnp.float32)
                   + bl2[...])
        inv[...] = iv

    return pl.pallas_call(
        body,
        grid=(grid,),
        in_specs=[
            pl.BlockSpec((NC, r, f_in), lambda i: (0, i, 0)),
            pl.BlockSpec((NC, r, 1), lambda i: (0, i, 0)),
            pl.BlockSpec((r, f_in), lambda i: (i, 0)),
            pl.BlockSpec((f_in, hid), lambda i: (0, 0)),
            pl.BlockSpec((1, hid), lambda i: (0, 0)),
            pl.BlockSpec((f_in, hid), lambda i: (0, 0)),
            pl.BlockSpec((hid, f_out), lambda i: (0, 0)),
            pl.BlockSpec((hid, f_out), lambda i: (0, 0)),
            pl.BlockSpec((1, f_out), lambda i: (0, 0)),
        ],
        out_specs=[
            pl.BlockSpec((r, f_out), lambda i: (i, 0)),
            pl.BlockSpec((r, f_out), lambda i: (i, 0)),
            pl.BlockSpec((r, 1), lambda i: (i, 0)),
        ],
        out_shape=[
            jax.ShapeDtypeStruct((n, f_out), jnp.float32),
            jax.ShapeDtypeStruct((n, f_out), jnp.float32),
            jax.ShapeDtypeStruct((n, 1), jnp.float32),
        ],
    )


@functools.lru_cache(maxsize=None)
def _make_tc2(n, f_out, r):
    grid = n // r

    def body(part, inv, r2, out):
        out[...] = (part[0] + part[1]) * inv[...] + r2[...]

    return pl.pallas_call(
        body,
        grid=(grid,),
        in_specs=[
            pl.BlockSpec((NC, r, f_out), lambda i: (0, i, 0)),
            pl.BlockSpec((r, 1), lambda i: (i, 0)),
            pl.BlockSpec((r, f_out), lambda i: (i, 0)),
        ],
        out_specs=pl.BlockSpec((r, f_out), lambda i: (i, 0)),
        out_shape=jax.ShapeDtypeStruct((n, f_out), jnp.float32),
    )


def kernel(x, edge_index, W_l1, b_l1, W_r1, W_l2, b_l2, W_r2):
    n, f_in = x.shape
    e = edge_index.shape[1]
    hid = W_l1.shape[1]
    f_out = W_l2.shape[1]

    edges = edge_index.reshape(2, e // K, K).transpose(1, 0, 2)
    cinit = jnp.concatenate(
        [jnp.ones((K, 1), jnp.float32), jnp.zeros((K, 1), jnp.float32)])

    part1, pcnt = _make_sc_scatter(n, e, f_in, True)(x, edges, cinit)
    p, r2, inv = _make_tc1(n, f_in, hid, f_out, 400)(
        part1, pcnt, x, W_l1, b_l1.reshape(1, hid), W_r1, W_l2, W_r2,
        b_l2.reshape(1, f_out))
    part2 = _make_sc_scatter(n, e, f_out, False)(p, edges)
    out = _make_tc2(n, f_out, 400)(part2, inv, r2)
    return out


# speedup vs baseline: 15.6607x; 1.0034x over previous
import functools

import jax
import jax.numpy as jnp
from jax import lax
from jax.experimental import pallas as pl
from jax.experimental.pallas import tpu as pltpu
from jax.experimental.pallas import tpu_sc as plsc

NC = 2
NS = 16
NW = NC * NS
K = 80


SPMEM_BUDGET = 2097151 * 4


@functools.lru_cache(maxsize=None)
def _make_sc_scatter(n, e, d, with_cnt):
    assert e % (NW * K) == 0 and n % K == 0 and d % 16 == 0
    epw = e // NW
    nch = epw // K
    nrc = n // K

    cnt_bytes = (n * 64 + NS * 2 * 64 * K) if with_cnt else 0
    nbuf = 2
    for cand in (3, 4):
        if n * d * 4 + NS * (cand * K * d * 4 + 2 * cand * 2 * K * 4) \
                + cnt_bytes < SPMEM_BUDGET:
            nbuf = cand
    ir = 2 * nbuf
    assert nch > ir

    mesh = plsc.VectorSubcoreMesh(core_axis_name="c", subcore_axis_name="s")

    out_type = [jax.ShapeDtypeStruct((NC, n, d), jnp.float32)]
    cnt_scratch = []
    if with_cnt:
        out_type.append(jax.ShapeDtypeStruct((NC, n, 16), jnp.float32))
        cnt_scratch = [
            pltpu.VMEM((K, 16), jnp.float32),
            pltpu.VMEM((K, 16), jnp.float32),
            pltpu.VMEM_SHARED((n, 16), jnp.float32),
        ]

    @functools.partial(
        pl.kernel,
        out_type=tuple(out_type) if with_cnt else out_type[0],
        mesh=mesh,
        scratch_types=[
            *[pltpu.VMEM((2, K), jnp.int32) for _ in range(ir)],
            *[pltpu.VMEM((K, d), jnp.float32) for _ in range(nbuf)],
            pltpu.VMEM_SHARED((n, d), jnp.float32),
            *cnt_scratch,
            *[pltpu.SemaphoreType.DMA for _ in range(ir + nbuf)],
        ],
        compiler_params=pltpu.CompilerParams(use_tc_tiling_on_sc=False),
    )
    def sc_kernel(*args):
        it = iter(args)
        rows_hbm = next(it)
        edges_hbm = next(it)
        part_hbm = next(it)
        pcnt_hbm = next(it) if with_cnt else None
        ix = [next(it) for _ in range(ir)]
        rows_v = [next(it) for _ in range(nbuf)]
        acc = next(it)
        if with_cnt:
            ones_v, zcol_v, acc1 = next(it), next(it), next(it)
        isem = [next(it) for _ in range(ir)]
        gsem = [next(it) for _ in range(nbuf)]

        c = lax.axis_index("c")
        s = lax.axis_index("s")
        wid = s * NC + c
        ch0 = wid * nch

        def ixload(i, a):
            return pltpu.make_async_copy(edges_hbm.at[ch0 + i], ix[a], isem[a])

        def gather(b, a):
            return pltpu.make_async_copy(
                rows_hbm.at[ix[a].at[0]], rows_v[b], gsem[b])

        def zrow(i, carry):
            for j in range(d // 16):
                rows_v[0][i, pl.ds(j * 16, 16)] = jnp.zeros((16,), jnp.float32)
            if with_cnt:
                ones_v[i, pl.ds(0, 16)] = jnp.ones((16,), jnp.float32)
                zcol_v[i, pl.ds(0, 16)] = jnp.zeros((16,), jnp.float32)
            return carry
        lax.fori_loop(0, K, zrow, 0)

        for a in range(ir):
            ixload(a, a).start()

        def zacc(i, carry):
            t = s + i * NS
            @pl.when(t < nrc)
            def _():
                pltpu.sync_copy(rows_v[0], acc.at[pl.ds(t * K, K)])
                if with_cnt:
                    pltpu.sync_copy(zcol_v, acc1.at[pl.ds(t * K, K)])
            return carry
        lax.fori_loop(0, (nrc + NS - 1) // NS, zacc, 0)
        plsc.subcore_barrier()

        for q in range(nbuf):
            ixload(q, q).wait()
            gather(q, q).start()

        def step(i, q, tail):
            b, a = q % nbuf, q % ir
            gather(b, a).wait()
            pltpu.sync_copy(rows_v[b], acc.at[ix[a].at[1]], add=True)
            if with_cnt:
                pltpu.sync_copy(ones_v, acc1.at[ix[a].at[1]], add=True)

            def refill():
                ixload(i + ir, a).start()
            def advance():
                a2 = (q + nbuf) % ir
                ixload(i + nbuf, a2).wait()
                gather(b, a2).start()
            if tail:
                if i + ir < nch:
                    refill()
                if i + nbuf < nch:
                    advance()
            else:
                pl.when(i + ir < nch)(refill)
                pl.when(i + nbuf < nch)(advance)

        def outer(j, carry):
            for q in range(ir):
                step(j * ir + q, q, False)
            return carry
        lax.fori_loop(0, nch // ir, outer, 0)
        for i in range((nch // ir) * ir, nch):
            step(i, i % ir, True)
        plsc.subcore_barrier()

        def cout(i, carry):
            t = s + i * NS
            @pl.when(t < nrc)
            def _():
                pltpu.sync_copy(acc.at[pl.ds(t * K, K)],
                                part_hbm.at[c, pl.ds(t * K, K)])
                if with_cnt:
                    pltpu.sync_copy(acc1.at[pl.ds(t * K, K)],
                                    pcnt_hbm.at[c, pl.ds(t * K, K)])
            return carry
        lax.fori_loop(0, (nrc + NS - 1) // NS, cout, 0)

    return sc_kernel


@functools.lru_cache(maxsize=None)
def _make_tc1(n, f_in, hid, f_out, r):
    grid = n // r

    def body(part, pcnt, x, wl1, bl1, wr1, wl2, wr2, bl2, p, r2, inv):
        a = part[0] + part[1]
        cnt = pcnt[0, :, 0:1] + pcnt[1, :, 0:1]
        iv = 1.0 / jnp.maximum(cnt, 1.0)
        mean = a * iv
        h = jnp.maximum(
            jnp.dot(mean, wl1[...], preferred_element_type=jnp.float32)
            + bl1[...]
            + jnp.dot(x[...], wr1[...], preferred_element_type=jnp.float32),
            0.0)
        p[...] = jnp.dot(h, wl2[...], preferred_element_type=jnp.float32)
        r2[...] = (jnp.dot(h, wr2[...], preferred_element_type=jnp.float32)
                   + bl2[...])
        inv[...] = iv

    return pl.pallas_call(
        body,
        grid=(grid,),
        in_specs=[
            pl.BlockSpec((NC, r, f_in), lambda i: (0, i, 0)),
            pl.BlockSpec((NC, r, 16), lambda i: (0, i, 0)),
            pl.BlockSpec((r, f_in), lambda i: (i, 0)),
            pl.BlockSpec((f_in, hid), lambda i: (0, 0)),
            pl.BlockSpec((1, hid), lambda i: (0, 0)),
            pl.BlockSpec((f_in, hid), lambda i: (0, 0)),
            pl.BlockSpec((hid, f_out), lambda i: (0, 0)),
            pl.BlockSpec((hid, f_out), lambda i: (0, 0)),
            pl.BlockSpec((1, f_out), lambda i: (0, 0)),
        ],
        out_specs=[
            pl.BlockSpec((r, f_out), lambda i: (i, 0)),
            pl.BlockSpec((r, f_out), lambda i: (i, 0)),
            pl.BlockSpec((r, 1), lambda i: (i, 0)),
        ],
        out_shape=[
            jax.ShapeDtypeStruct((n, f_out), jnp.float32),
            jax.ShapeDtypeStruct((n, f_out), jnp.float32),
            jax.ShapeDtypeStruct((n, 1), jnp.float32),
        ],
    )


@functools.lru_cache(maxsize=None)
def _make_tc2(n, f_out, r):
    grid = n // r

    def body(part, inv, r2, out):
        out[...] = (part[0] + part[1]) * inv[...] + r2[...]

    return pl.pallas_call(
        body,
        grid=(grid,),
        in_specs=[
            pl.BlockSpec((NC, r, f_out), lambda i: (0, i, 0)),
            pl.BlockSpec((r, 1), lambda i: (i, 0)),
            pl.BlockSpec((r, f_out), lambda i: (i, 0)),
        ],
        out_specs=pl.BlockSpec((r, f_out), lambda i: (i, 0)),
        out_shape=jax.ShapeDtypeStruct((n, f_out), jnp.float32),
    )


def kernel(x, edge_index, W_l1, b_l1, W_r1, W_l2, b_l2, W_r2):
    n, f_in = x.shape
    e = edge_index.shape[1]
    hid = W_l1.shape[1]
    f_out = W_l2.shape[1]

    edges = edge_index.reshape(2, e // K, K).transpose(1, 0, 2)

    part1, pcnt = _make_sc_scatter(n, e, f_in, True)(x, edges)
    p, r2, inv = _make_tc1(n, f_in, hid, f_out, 400)(
        part1, pcnt, x, W_l1, b_l1.reshape(1, hid), W_r1, W_l2, W_r2,
        b_l2.reshape(1, f_out))
    part2 = _make_sc_scatter(n, e, f_out, False)(p, edges)
    out = _make_tc2(n, f_out, 400)(part2, inv, r2)
    return out
